# baseline (speedup 1.0000x reference)
import numpy as np
import jax
import jax.numpy as jnp
from jax import lax
from jax.experimental import pallas as pl

_V = 100000
_R = 8
_NROWS = 512

_ROT = ((13, 15, 26, 6), (17, 29, 16, 24),
        (13, 15, 26, 6), (17, 29, 16, 24),
        (13, 15, 26, 6))

_KS = (np.uint32(0), np.uint32(42),
       np.uint32(np.uint32(42) ^ np.uint32(0x1BD11BDA)))


def _rotl(x, r):
    return (x << np.uint32(r)) | (x >> np.uint32(32 - r))


def _gumbel_bits(flat_u32):
    x0 = jnp.zeros_like(flat_u32) + _KS[0]
    x1 = flat_u32 + _KS[1]
    for g in range(5):
        for r in _ROT[g]:
            x0 = x0 + x1
            x1 = _rotl(x1, r) ^ x0
        x0 = x0 + _KS[(g + 1) % 3]
        x1 = x1 + _KS[(g + 2) % 3] + np.uint32(g + 1)
    return x0 ^ x1


def _body(lg_ref, samp_ref, lp_ref):
    i = pl.program_id(0)
    lg = lg_ref[...]

    col = lax.broadcasted_iota(jnp.int32, (_R, _V), 1)
    row = lax.broadcasted_iota(jnp.int32, (_R, _V), 0) + i * _R
    flat = (row * _V + col).astype(jnp.uint32)

    bits = _gumbel_bits(flat)
    fl = lax.bitcast_convert_type(
        (bits >> np.uint32(9)) | np.uint32(0x3F800000), jnp.float32) - 1.0
    tiny = np.float32(np.finfo(np.float32).tiny)
    u = jnp.maximum(fl + tiny, tiny)
    gum = -jnp.log(-jnp.log(u))
    t = gum + lg

    m_t = jnp.max(t, axis=-1, keepdims=True)
    samp = jnp.min(jnp.where(t == m_t, col, _V), axis=-1, keepdims=True)

    m_l = jnp.max(lg, axis=-1, keepdims=True)
    s = jnp.sum(jnp.exp(lg - m_l), axis=-1, keepdims=True)
    chosen = jnp.sum(jnp.where(col == samp, lg, 0.0), axis=-1, keepdims=True)
    lp_row = (chosen - m_l) - jnp.log(s)

    samp_ref[0] = samp
    lp_ref[0] = jnp.full((_R, 1), jnp.sum(lp_row), jnp.float32)


def kernel(logits):
    lg = logits.reshape(_NROWS, _V)
    nblk = _NROWS // _R
    samp, lp = pl.pallas_call(
        _body,
        grid=(nblk,),
        in_specs=[pl.BlockSpec((_R, _V), lambda i: (i, 0))],
        out_specs=[
            pl.BlockSpec((1, _R, 1), lambda i: (i, 0, 0)),
            pl.BlockSpec((1, _R, 1), lambda i: (i, 0, 0)),
        ],
        out_shape=[
            jax.ShapeDtypeStruct((nblk, _R, 1), jnp.int32),
            jax.ShapeDtypeStruct((nblk, _R, 1), jnp.float32),
        ],
    )(lg)
    sample = samp[..., 0]
    log_prob = lp[:, 0, 0]
    return sample, log_prob
